# baseline (speedup 1.0000x reference)
import functools

import jax
import jax.numpy as jnp
from jax import lax
from jax.experimental import pallas as pl
from jax.experimental.pallas import tpu as pltpu
from jax.experimental.pallas import tpu_sc as plsc

NC = 2
NS = 16
NW = NC * NS
CHUNK = 128
PAD_ROWS = 16


def _mesh():
    return plsc.VectorSubcoreMesh(core_axis_name="c", subcore_axis_name="s")


def _make_deg_kernel(n_chunks: int, nr: int):
    @functools.partial(
        pl.kernel,
        mesh=_mesh(),
        out_type=(
            jax.ShapeDtypeStruct((nr,), jnp.float32),
            jax.ShapeDtypeStruct((nr,), jnp.float32),
        ),
        scratch_types=[
            pltpu.VMEM((n_chunks, CHUNK), jnp.int32),
            pltpu.VMEM((CHUNK,), jnp.float32),
            pltpu.VMEM_SHARED((nr,), jnp.float32),
            pltpu.SemaphoreType.DMA,
        ],
    )
    def deg_k(colp3_hbm, zeros1_hbm, ones1_hbm, out0_hbm, out1_hbm,
              cidx_all, ones_v, deg_sh, sem):
        cid = lax.axis_index("c")
        sid = lax.axis_index("s")
        wid = cid * NS + sid

        @pl.when(sid == 0)
        def _zero():
            pltpu.sync_copy(zeros1_hbm, deg_sh)

        pltpu.sync_copy(ones1_hbm, ones_v)
        pltpu.sync_copy(colp3_hbm.at[wid], cidx_all)
        plsc.subcore_barrier()

        blk = 8

        def block(q, carry):
            for j in range(blk):
                pltpu.async_copy(
                    ones_v, deg_sh.at[cidx_all.at[blk * q + j]], sem, add=True)
            for j in range(blk):
                pltpu.make_async_copy(
                    ones_v, deg_sh.at[cidx_all.at[blk * q + j]], sem).wait()
            return carry

        lax.fori_loop(0, n_chunks // blk, block, 0)
        plsc.subcore_barrier()

        @pl.when(jnp.logical_and(sid == 0, cid == 0))
        def _out0():
            pltpu.sync_copy(deg_sh, out0_hbm)

        @pl.when(jnp.logical_and(sid == 0, cid == 1))
        def _out1():
            pltpu.sync_copy(deg_sh, out1_hbm)

    return deg_k


def _make_agg_kernel(n_nodes: int, d: int, chunk: int, n_chunks: int,
                     nr: int, r_tile: int):
    last_tile = n_nodes - (NS - 1) * r_tile

    nbuf = 3
    shift = 14
    mask = (1 << shift) - 1

    @functools.partial(
        pl.kernel,
        mesh=_mesh(),
        out_type=jax.ShapeDtypeStruct((NC, n_nodes, d), jnp.float32),
        scratch_types=[
            pltpu.VMEM((chunk, d), jnp.float32),
            pltpu.VMEM((chunk, d), jnp.float32),
            pltpu.VMEM((chunk, d), jnp.float32),
            pltpu.VMEM((n_chunks, chunk), jnp.int32),
            pltpu.VMEM((chunk,), jnp.int32),
            pltpu.VMEM((chunk,), jnp.int32),
            pltpu.VMEM((chunk,), jnp.int32),
            pltpu.VMEM((chunk,), jnp.int32),
            pltpu.VMEM((chunk,), jnp.int32),
            pltpu.VMEM((chunk,), jnp.int32),
            pltpu.VMEM_SHARED((nr, d), jnp.float32),
            pltpu.SemaphoreType.DMA,
            pltpu.SemaphoreType.DMA,
            pltpu.SemaphoreType.DMA,
            pltpu.SemaphoreType.DMA,
            pltpu.SemaphoreType.DMA,
            pltpu.SemaphoreType.DMA,
            pltpu.SemaphoreType.DMA,
            pltpu.SemaphoreType.DMA,
            pltpu.SemaphoreType.DMA,
        ],
    )
    def agg_k(g_hbm, pidx3_hbm, out_hbm,
              rbuf0, rbuf1, rbuf2, pidx_all, ridx0, ridx1, ridx2,
              cidx0, cidx1, cidx2, acc_sh,
              gsem0, gsem1, gsem2, hsem0, hsem1, hsem2,
              ssem0, ssem1, ssem2):
        cid = lax.axis_index("c")
        sid = lax.axis_index("s")
        wid = cid * NS + sid
        rbuf = (rbuf0, rbuf1, rbuf2)
        ridx = (ridx0, ridx1, ridx2)
        cidx = (cidx0, cidx1, cidx2)
        gsem = (gsem0, gsem1, gsem2)
        hsem = (hsem0, hsem1, hsem2)
        ssem = (ssem0, ssem1, ssem2)
        sp1 = chunk // 16 // 2 * 16
        sp2 = chunk - sp1
        pltpu.sync_copy(pidx3_hbm.at[wid], pidx_all)

        def unpack(ci, b):
            for c in range(chunk // 16):
                v = pidx_all[ci, pl.ds(c * 16, 16)]
                ridx[b][pl.ds(c * 16, 16)] = lax.shift_right_logical(v, shift)
                cidx[b][pl.ds(c * 16, 16)] = lax.bitwise_and(v, mask)

        def gather(b):
            pltpu.async_copy(g_hbm.at[ridx[b].at[pl.ds(0, sp1)]],
                             rbuf[b].at[pl.ds(0, sp1)], gsem[b])
            pltpu.async_copy(g_hbm.at[ridx[b].at[pl.ds(sp1, sp2)]],
                             rbuf[b].at[pl.ds(sp1, sp2)], hsem[b])

        def wait_gather(b):
            pltpu.make_async_copy(g_hbm.at[ridx[b].at[pl.ds(0, sp1)]],
                                  rbuf[b].at[pl.ds(0, sp1)], gsem[b]).wait()
            pltpu.make_async_copy(g_hbm.at[ridx[b].at[pl.ds(sp1, sp2)]],
                                  rbuf[b].at[pl.ds(sp1, sp2)], hsem[b]).wait()

        def scatter(b):
            pltpu.async_copy(rbuf[b], acc_sh.at[cidx[b]], ssem[b], add=True)

        def wait_scatter(b):
            pltpu.make_async_copy(rbuf[b], acc_sh.at[cidx[b]], ssem[b]).wait()

        unpack(0, 0)
        gather(0)
        unpack(1, 1)
        gather(1)

        zero16 = jnp.zeros((16,), jnp.float32)

        def zrow(i, carry):
            for c in range(d // 16):
                rbuf2[i, pl.ds(c * 16, 16)] = zero16
            return carry

        lax.fori_loop(0, chunk, zrow, 0)
        full_reps = r_tile // chunk
        for k in range(full_reps):
            pltpu.sync_copy(rbuf2,
                            acc_sh.at[pl.ds(sid * r_tile + k * chunk, chunk)])
        rem = r_tile - full_reps * chunk
        if rem:
            pltpu.sync_copy(
                rbuf2.at[pl.ds(0, rem)],
                acc_sh.at[pl.ds(sid * r_tile + full_reps * chunk, rem)])
        plsc.subcore_barrier()

        def outer(q, carry):
            for b in range(nbuf):
                i = nbuf * q + b
                bp = (b + 2) % nbuf
                wait_gather(b)
                scatter(b)

                @pl.when(jnp.logical_and(i >= 1, i + 2 < n_chunks))
                def _reclaim():
                    wait_scatter(bp)

                @pl.when(i + 2 < n_chunks)
                def _refill():
                    unpack(i + 2, bp)
                    gather(bp)
            return carry

        lax.fori_loop(0, n_chunks // nbuf, outer, 0)
        wait_scatter(0)
        wait_scatter(1)
        wait_scatter(2)
        plsc.subcore_barrier()

        @pl.when(sid < NS - 1)
        def _copy_full():
            pltpu.sync_copy(
                acc_sh.at[pl.ds(sid * r_tile, r_tile)],
                out_hbm.at[cid, pl.ds(sid * r_tile, r_tile)],
            )

        @pl.when(sid == NS - 1)
        def _copy_last():
            pltpu.sync_copy(
                acc_sh.at[pl.ds((NS - 1) * r_tile, last_tile)],
                out_hbm.at[cid, pl.ds((NS - 1) * r_tile, last_tile)],
            )

    return agg_k


def _tc_mm(x, w):
    n = x.shape[0]
    dout = w.shape[1]

    def body(x_ref, w_ref, h_ref):
        h_ref[...] = jnp.dot(x_ref[...], w_ref[...],
                             preferred_element_type=jnp.float32)

    return pl.pallas_call(
        body, out_shape=jax.ShapeDtypeStruct((n, dout), jnp.float32))(x, w)


def _tc_scale(h, d0, d1):
    n, dout = h.shape

    def body(h_ref, d0_ref, d1_ref, g_ref, dis_ref):
        deg = d0_ref[...] + d1_ref[...] + 1.0
        dis = lax.rsqrt(deg)
        g_ref[...] = dis * h_ref[...]
        dis_ref[...] = dis

    return pl.pallas_call(
        body,
        out_shape=(
            jax.ShapeDtypeStruct((n, dout), jnp.float32),
            jax.ShapeDtypeStruct((n, 1), jnp.float32),
        ),
    )(h, d0, d1)


def _tc_mid(a0, a1, g1, dis, b1, gamma, beta, mu, var, w2):
    n, d = g1.shape
    dout = w2.shape[1]

    def body(a0_ref, a1_ref, g1_ref, dis_ref, b1_ref, ga_ref, be_ref,
             mu_ref, var_ref, w2_ref, g2_ref):
        dis = dis_ref[...]
        y = dis * (a0_ref[...] + a1_ref[...] + g1_ref[...]) + b1_ref[...]
        y = (y - mu_ref[...]) * lax.rsqrt(var_ref[...] + 1e-5) * ga_ref[...]
        y = y + be_ref[...]
        y = jnp.maximum(y, 0.0)
        g2_ref[...] = dis * jnp.dot(y, w2_ref[...],
                                    preferred_element_type=jnp.float32)

    return pl.pallas_call(
        body,
        out_shape=jax.ShapeDtypeStruct((n, dout), jnp.float32),
    )(a0, a1, g1, dis, b1, gamma, beta, mu, var, w2)


def _tc_final(a0, a1, g2, dis, b2):
    n, d = g2.shape

    def body(a0_ref, a1_ref, g2_ref, dis_ref, b2_ref, o_ref):
        o_ref[...] = (dis_ref[...] * (a0_ref[...] + a1_ref[...] + g2_ref[...])
                      + b2_ref[...])

    return pl.pallas_call(
        body,
        out_shape=jax.ShapeDtypeStruct((n, d), jnp.float32),
    )(a0, a1, g2, dis, b2)


def kernel(x, edge_index, W1, b1, gamma, beta, bn_mean, bn_var, W2, b2):
    n, d = x.shape
    e = edge_index.shape[1]
    row = edge_index[0].astype(jnp.int32)
    col = edge_index[1].astype(jnp.int32)

    def _padded(chunk, mult):
        base = -(-e // (NW * chunk))
        nch = max(2 * mult, (base + mult - 1) // mult * mult)
        pad = NW * nch * chunk - e
        ar = jnp.arange(pad, dtype=jnp.int32)
        rp = jnp.concatenate([row, (ar * 97) % n])
        cp = jnp.concatenate([col, n + (ar % PAD_ROWS)])
        return rp.reshape(NW, nch, chunk), cp.reshape(NW, nch, chunk), nch

    _, colp3, n_chunks_deg = _padded(CHUNK, 8)
    ch_agg = 80
    rowp3, colp3a, n_chunks_agg = _padded(ch_agg, 3)
    pidx3 = rowp3 * (1 << 14) + colp3a

    r_tile = ((n + PAD_ROWS + NS - 1) // NS + 7) // 8 * 8
    nr = NS * r_tile

    zeros1 = jnp.zeros((nr,), jnp.float32)
    ones1 = jnp.ones((CHUNK,), jnp.float32)

    h1 = _tc_mm(x, W1)
    deg0, deg1 = _make_deg_kernel(n_chunks_deg, nr)(colp3, zeros1, ones1)
    d0 = deg0[:n, None]
    d1 = deg1[:n, None]

    g1, dis = _tc_scale(h1, d0, d1)

    agg = _make_agg_kernel(n, d, ch_agg, n_chunks_agg, nr, r_tile)
    a1 = agg(g1, pidx3)
    g2 = _tc_mid(a1[0], a1[1], g1, dis, b1[None, :], gamma[None, :],
                 beta[None, :], bn_mean[None, :], bn_var[None, :], W2)
    a2 = agg(g2, pidx3)
    return _tc_final(a2[0], a2[1], g2, dis, b2[None, :])


# speedup vs baseline: 1.0014x; 1.0014x over previous
import functools

import jax
import jax.numpy as jnp
from jax import lax
from jax.experimental import pallas as pl
from jax.experimental.pallas import tpu as pltpu
from jax.experimental.pallas import tpu_sc as plsc

NC = 2
NS = 16
NW = NC * NS
CHUNK = 128
PAD_ROWS = 16


def _mesh():
    return plsc.VectorSubcoreMesh(core_axis_name="c", subcore_axis_name="s")


def _make_deg_kernel(n_chunks: int, nr: int):
    @functools.partial(
        pl.kernel,
        mesh=_mesh(),
        out_type=(
            jax.ShapeDtypeStruct((nr,), jnp.float32),
            jax.ShapeDtypeStruct((nr,), jnp.float32),
        ),
        scratch_types=[
            pltpu.VMEM((n_chunks, CHUNK), jnp.int32),
            pltpu.VMEM((CHUNK,), jnp.float32),
            pltpu.VMEM_SHARED((nr,), jnp.float32),
            pltpu.SemaphoreType.DMA,
        ],
    )
    def deg_k(colp3_hbm, zeros1_hbm, ones1_hbm, out0_hbm, out1_hbm,
              cidx_all, ones_v, deg_sh, sem):
        cid = lax.axis_index("c")
        sid = lax.axis_index("s")
        wid = cid * NS + sid

        @pl.when(sid == 0)
        def _zero():
            pltpu.sync_copy(zeros1_hbm, deg_sh)

        pltpu.sync_copy(ones1_hbm, ones_v)
        pltpu.sync_copy(colp3_hbm.at[wid], cidx_all)
        plsc.subcore_barrier()

        blk = 8

        def block(q, carry):
            for j in range(blk):
                pltpu.async_copy(
                    ones_v, deg_sh.at[cidx_all.at[blk * q + j]], sem, add=True)
            for j in range(blk):
                pltpu.make_async_copy(
                    ones_v, deg_sh.at[cidx_all.at[blk * q + j]], sem).wait()
            return carry

        lax.fori_loop(0, n_chunks // blk, block, 0)
        plsc.subcore_barrier()

        @pl.when(jnp.logical_and(sid == 0, cid == 0))
        def _out0():
            pltpu.sync_copy(deg_sh, out0_hbm)

        @pl.when(jnp.logical_and(sid == 0, cid == 1))
        def _out1():
            pltpu.sync_copy(deg_sh, out1_hbm)

    return deg_k


def _make_agg_kernel(n_nodes: int, d: int, chunk: int, n_chunks: int,
                     nr: int, r_tile: int):
    last_tile = n_nodes - (NS - 1) * r_tile

    nbuf = 3
    shift = 14
    mask = (1 << shift) - 1

    @functools.partial(
        pl.kernel,
        mesh=_mesh(),
        out_type=jax.ShapeDtypeStruct((NC, n_nodes, d), jnp.float32),
        scratch_types=[
            pltpu.VMEM((chunk, d), jnp.float32),
            pltpu.VMEM((chunk, d), jnp.float32),
            pltpu.VMEM((chunk, d), jnp.float32),
            pltpu.VMEM((n_chunks, chunk), jnp.int32),
            pltpu.VMEM((chunk,), jnp.int32),
            pltpu.VMEM((chunk,), jnp.int32),
            pltpu.VMEM((chunk,), jnp.int32),
            pltpu.VMEM((chunk,), jnp.int32),
            pltpu.VMEM((chunk,), jnp.int32),
            pltpu.VMEM((chunk,), jnp.int32),
            pltpu.VMEM_SHARED((nr, d), jnp.float32),
            pltpu.SemaphoreType.DMA,
            pltpu.SemaphoreType.DMA,
            pltpu.SemaphoreType.DMA,
            pltpu.SemaphoreType.DMA,
            pltpu.SemaphoreType.DMA,
            pltpu.SemaphoreType.DMA,
            pltpu.SemaphoreType.DMA,
            pltpu.SemaphoreType.DMA,
            pltpu.SemaphoreType.DMA,
        ],
    )
    def agg_k(g_hbm, pidx3_hbm, out_hbm,
              rbuf0, rbuf1, rbuf2, pidx_all, ridx0, ridx1, ridx2,
              cidx0, cidx1, cidx2, acc_sh,
              gsem0, gsem1, gsem2, hsem0, hsem1, hsem2,
              ssem0, ssem1, ssem2):
        cid = lax.axis_index("c")
        sid = lax.axis_index("s")
        wid = cid * NS + sid
        rbuf = (rbuf0, rbuf1, rbuf2)
        ridx = (ridx0, ridx1, ridx2)
        cidx = (cidx0, cidx1, cidx2)
        gsem = (gsem0, gsem1, gsem2)
        hsem = (hsem0, hsem1, hsem2)
        ssem = (ssem0, ssem1, ssem2)
        sp1 = chunk // 16 // 2 * 16
        sp2 = chunk - sp1
        pltpu.sync_copy(pidx3_hbm.at[wid], pidx_all)

        def unpack(ci, b):
            for c in range(chunk // 16):
                v = pidx_all[ci, pl.ds(c * 16, 16)]
                ridx[b][pl.ds(c * 16, 16)] = lax.shift_right_logical(v, shift)
                cidx[b][pl.ds(c * 16, 16)] = lax.bitwise_and(v, mask)

        def gather(b):
            pltpu.async_copy(g_hbm.at[ridx[b].at[pl.ds(0, sp1)]],
                             rbuf[b].at[pl.ds(0, sp1)], gsem[b])
            pltpu.async_copy(g_hbm.at[ridx[b].at[pl.ds(sp1, sp2)]],
                             rbuf[b].at[pl.ds(sp1, sp2)], hsem[b])

        def wait_gather(b):
            pltpu.make_async_copy(g_hbm.at[ridx[b].at[pl.ds(0, sp1)]],
                                  rbuf[b].at[pl.ds(0, sp1)], gsem[b]).wait()
            pltpu.make_async_copy(g_hbm.at[ridx[b].at[pl.ds(sp1, sp2)]],
                                  rbuf[b].at[pl.ds(sp1, sp2)], hsem[b]).wait()

        def scatter(b):
            pltpu.async_copy(rbuf[b], acc_sh.at[cidx[b]], ssem[b], add=True)

        def wait_scatter(b):
            pltpu.make_async_copy(rbuf[b], acc_sh.at[cidx[b]], ssem[b]).wait()

        unpack(0, 0)
        gather(0)
        unpack(1, 1)
        gather(1)

        zero16 = jnp.zeros((16,), jnp.float32)

        def zrow(i, carry):
            for c in range(d // 16):
                rbuf2[i, pl.ds(c * 16, 16)] = zero16
            return carry

        lax.fori_loop(0, chunk, zrow, 0)
        full_reps = r_tile // chunk
        for k in range(full_reps):
            pltpu.sync_copy(rbuf2,
                            acc_sh.at[pl.ds(sid * r_tile + k * chunk, chunk)])
        rem = r_tile - full_reps * chunk
        if rem:
            pltpu.sync_copy(
                rbuf2.at[pl.ds(0, rem)],
                acc_sh.at[pl.ds(sid * r_tile + full_reps * chunk, rem)])
        plsc.subcore_barrier()

        def outer(q, carry):
            for b in range(nbuf):
                i = nbuf * q + b
                bp = (b + 2) % nbuf
                wait_gather(b)
                scatter(b)

                @pl.when(jnp.logical_and(i >= 1, i + 2 < n_chunks))
                def _reclaim():
                    wait_scatter(bp)

                @pl.when(i + 2 < n_chunks)
                def _refill():
                    unpack(i + 2, bp)
                    gather(bp)
            return carry

        lax.fori_loop(0, n_chunks // nbuf, outer, 0)
        wait_scatter(0)
        wait_scatter(1)
        wait_scatter(2)
        plsc.subcore_barrier()

        @pl.when(sid < NS - 1)
        def _copy_full():
            pltpu.sync_copy(
                acc_sh.at[pl.ds(sid * r_tile, r_tile)],
                out_hbm.at[cid, pl.ds(sid * r_tile, r_tile)],
            )

        @pl.when(sid == NS - 1)
        def _copy_last():
            pltpu.sync_copy(
                acc_sh.at[pl.ds((NS - 1) * r_tile, last_tile)],
                out_hbm.at[cid, pl.ds((NS - 1) * r_tile, last_tile)],
            )

    return agg_k


def _tc_scale_mm(x, w, d0, d1):
    n, din = x.shape
    dout = w.shape[1]

    def body(x_ref, w_ref, d0_ref, d1_ref, g_ref, dis_ref):
        deg = d0_ref[...] + d1_ref[...] + 1.0
        dis = lax.rsqrt(deg)
        h = jnp.dot(x_ref[...], w_ref[...], preferred_element_type=jnp.float32)
        g_ref[...] = dis * h
        dis_ref[...] = dis

    return pl.pallas_call(
        body,
        out_shape=(
            jax.ShapeDtypeStruct((n, dout), jnp.float32),
            jax.ShapeDtypeStruct((n, 1), jnp.float32),
        ),
    )(x, w, d0, d1)


def _tc_mid(a0, a1, g1, dis, b1, gamma, beta, mu, var, w2):
    n, d = g1.shape
    dout = w2.shape[1]

    def body(a0_ref, a1_ref, g1_ref, dis_ref, b1_ref, ga_ref, be_ref,
             mu_ref, var_ref, w2_ref, g2_ref):
        dis = dis_ref[...]
        y = dis * (a0_ref[...] + a1_ref[...] + g1_ref[...]) + b1_ref[...]
        y = (y - mu_ref[...]) * lax.rsqrt(var_ref[...] + 1e-5) * ga_ref[...]
        y = y + be_ref[...]
        y = jnp.maximum(y, 0.0)
        g2_ref[...] = dis * jnp.dot(y, w2_ref[...],
                                    preferred_element_type=jnp.float32)

    return pl.pallas_call(
        body,
        out_shape=jax.ShapeDtypeStruct((n, dout), jnp.float32),
    )(a0, a1, g1, dis, b1, gamma, beta, mu, var, w2)


def _tc_final(a0, a1, g2, dis, b2):
    n, d = g2.shape

    def body(a0_ref, a1_ref, g2_ref, dis_ref, b2_ref, o_ref):
        o_ref[...] = (dis_ref[...] * (a0_ref[...] + a1_ref[...] + g2_ref[...])
                      + b2_ref[...])

    return pl.pallas_call(
        body,
        out_shape=jax.ShapeDtypeStruct((n, d), jnp.float32),
    )(a0, a1, g2, dis, b2)


def kernel(x, edge_index, W1, b1, gamma, beta, bn_mean, bn_var, W2, b2):
    n, d = x.shape
    e = edge_index.shape[1]
    row = edge_index[0].astype(jnp.int32)
    col = edge_index[1].astype(jnp.int32)

    def _padded(chunk, mult):
        base = -(-e // (NW * chunk))
        nch = max(2 * mult, (base + mult - 1) // mult * mult)
        pad = NW * nch * chunk - e
        ar = jnp.arange(pad, dtype=jnp.int32)
        rp = jnp.concatenate([row, (ar * 97) % n])
        cp = jnp.concatenate([col, n + (ar % PAD_ROWS)])
        return rp.reshape(NW, nch, chunk), cp.reshape(NW, nch, chunk), nch

    _, colp3, n_chunks_deg = _padded(CHUNK, 8)
    ch_agg = 80
    rowp3, colp3a, n_chunks_agg = _padded(ch_agg, 3)
    pidx3 = rowp3 * (1 << 14) + colp3a

    r_tile = ((n + PAD_ROWS + NS - 1) // NS + 7) // 8 * 8
    nr = NS * r_tile

    zeros1 = jnp.zeros((nr,), jnp.float32)
    ones1 = jnp.ones((CHUNK,), jnp.float32)

    deg0, deg1 = _make_deg_kernel(n_chunks_deg, nr)(colp3, zeros1, ones1)
    d0 = deg0[:n, None]
    d1 = deg1[:n, None]

    g1, dis = _tc_scale_mm(x, W1, d0, d1)

    agg = _make_agg_kernel(n, d, ch_agg, n_chunks_agg, nr, r_tile)
    a1 = agg(g1, pidx3)
    g2 = _tc_mid(a1[0], a1[1], g1, dis, b1[None, :], gamma[None, :],
                 beta[None, :], bn_mean[None, :], bn_var[None, :], W2)
    a2 = agg(g2, pidx3)
    return _tc_final(a2[0], a2[1], g2, dis, b2[None, :])
